# baseline (speedup 1.0000x reference)
import jax
import jax.numpy as jnp
from jax import lax
from jax.experimental import pallas as pl
from jax.experimental.pallas import tpu as pltpu
from jax.experimental.pallas import tpu_sc as plsc

_C = 80
_REG = 33
_D = 4 * _REG
_N = 300
_M = 16
_B = 8
_TASKS = 48
_LANES = 16
_CHUNKS = 19
_NPAD = _CHUNKS * _LANES
_BIG = 1e30
_LN2 = 0.6931471805599453


def _iota():
    return lax.iota(jnp.int32, _LANES)


def _splat_i(x):
    return jnp.zeros((_LANES,), jnp.int32) + x


def _sigmoid(x):
    return 1.0 / (1.0 + jnp.exp(-x))


def _log1p_small(z):
    s = z / (2.0 + z)
    s2 = s * s
    return s * (2.0 + s2 * (2.0 / 3.0 + s2 * (2.0 / 5.0 + s2 * (
        2.0 / 7.0 + s2 * (2.0 / 9.0 + s2 * (2.0 / 11.0))))))


def _log(x):
    xi = lax.bitcast_convert_type(x, jnp.int32)
    e = jnp.right_shift(xi, 23) - 127
    m = lax.bitcast_convert_type(
        jnp.bitwise_or(jnp.bitwise_and(xi, 0x7FFFFF), 0x3F800000), jnp.float32)
    big = m > 1.4142135623730951
    m = jnp.where(big, m * 0.5, m)
    ef = (e + big.astype(jnp.int32)).astype(jnp.float32)
    s = (m - 1.0) / (m + 1.0)
    s2 = s * s
    poly = s * (2.0 + s2 * (2.0 / 3.0 + s2 * (2.0 / 5.0 + s2 * (
        2.0 / 7.0 + s2 * (2.0 / 9.0)))))
    return ef * _LN2 + poly


def _sc_body(plog_hbm, pbox_hbm, pcor_hbm, alog_hbm, abox_hbm, acor_hbm,
             labels_hbm, tbox_hbm, out_hbm,
             logit_v, box_v, corner_v, cost_v, pen_v, lab_v, tbox_v,
             part_v, out_v, shared_sp, red_v):
    core = lax.axis_index("c")
    sub = lax.axis_index("s")
    gwid = sub * 2 + core
    ii = _iota()
    zf = jnp.zeros((_LANES,), jnp.float32)

    part_v[pl.ds(0, _LANES)] = zf
    part_v[pl.ds(_LANES, _LANES)] = zf

    def do_task(t):
        lyr = t // _B
        b = t - lyr * _B

        @pl.when(t < _B)
        def _load_final():
            pltpu.sync_copy(plog_hbm.at[t], logit_v)
            pltpu.sync_copy(pbox_hbm.at[t], box_v)
            pltpu.sync_copy(pcor_hbm.at[t], corner_v)

        @pl.when(t >= _B)
        def _load_aux():
            ta = t - _B
            pltpu.sync_copy(alog_hbm.at[ta], logit_v)
            pltpu.sync_copy(abox_hbm.at[ta], box_v)
            pltpu.sync_copy(acor_hbm.at[ta], corner_v)

        pltpu.sync_copy(labels_hbm.at[b], lab_v)
        pltpu.sync_copy(tbox_hbm.at[b], tbox_v)

        @pl.loop(0, _CHUNKS)
        def _cost_chunk(c):
            n_idx = c * _LANES + ii
            valid = n_idx < _N
            nc = jnp.minimum(n_idx, _N - 1)
            nc4 = nc * 4
            bx0 = plsc.load_gather(box_v, [nc4])
            by0 = plsc.load_gather(box_v, [nc4 + 1])
            bx1 = plsc.load_gather(box_v, [nc4 + 2])
            by1 = plsc.load_gather(box_v, [nc4 + 3])

            @pl.loop(0, _M)
            def _cost_m(m):
                ms4 = _splat_i(m * 4)
                tx0 = plsc.load_gather(tbox_v, [ms4])
                ty0 = plsc.load_gather(tbox_v, [ms4 + 1])
                tx1 = plsc.load_gather(tbox_v, [ms4 + 2])
                ty1 = plsc.load_gather(tbox_v, [ms4 + 3])
                labm = plsc.load_gather(lab_v, [_splat_i(m)])
                lg = plsc.load_gather(logit_v, [nc, labm])
                cb = (jnp.abs(bx0 - tx0) + jnp.abs(by0 - ty0)
                      + jnp.abs(bx1 - tx1) + jnp.abs(by1 - ty1))
                cv = 5.0 * cb - _sigmoid(lg)
                cv = jnp.where(valid, cv, _BIG)
                cost_v[pl.ds(m * _NPAD + c * _LANES, _LANES)] = cv

        @pl.loop(0, _CHUNKS)
        def _pen_init(c):
            pen_v[pl.ds(c * _LANES, _LANES)] = zf

        def greedy_body(j, srcvec):
            def chunk_body(c, mc):
                mv, mi = mc
                v = (cost_v[pl.ds(j * _NPAD + c * _LANES, _LANES)]
                     + pen_v[pl.ds(c * _LANES, _LANES)])
                idxv = c * _LANES + ii
                take = v < mv
                return (jnp.where(take, v, mv), jnp.where(take, idxv, mi))

            mv, mi = lax.fori_loop(
                0, _CHUNKS, chunk_body,
                (jnp.full((_LANES,), 1e33, jnp.float32),
                 jnp.zeros((_LANES,), jnp.int32)))
            gmin = jnp.min(mv)
            cand = jnp.where(mv == gmin, mi, jnp.int32(2**31 - 1))
            gidx = jnp.min(cand)
            plsc.store_scatter(pen_v, [_splat_i(gidx)],
                               jnp.full((_LANES,), _BIG, jnp.float32),
                               mask=ii == 0)
            return jnp.where(ii == j, gidx, srcvec)

        srcvec = lax.fori_loop(0, _M, greedy_body,
                               jnp.zeros((_LANES,), jnp.int32))

        def vfl_col(cidx, acc):
            l = plsc.load_gather(logit_v, [srcvec, _splat_i(cidx)])
            p = _sigmoid(l)
            bce0 = jnp.maximum(l, 0.0) + _log1p_small(jnp.exp(-jnp.abs(l)))
            return acc + (0.75 * p * p) * bce0

        acc0 = lax.fori_loop(0, _C, vfl_col, zf)
        labvec = lab_v[...]
        lt = plsc.load_gather(logit_v, [srcvec, labvec])
        pt = _sigmoid(lt)
        bce0t = jnp.maximum(lt, 0.0) + _log1p_small(jnp.exp(-jnp.abs(lt)))
        vfl_vec = acc0 + (bce0t - lt) - (0.75 * pt * pt) * bce0t
        vfl_s = jnp.sum(vfl_vec)

        src4 = srcvec * 4
        mbx0 = plsc.load_gather(box_v, [src4])
        mby0 = plsc.load_gather(box_v, [src4 + 1])
        mbx1 = plsc.load_gather(box_v, [src4 + 2])
        mby1 = plsc.load_gather(box_v, [src4 + 3])
        cx = (mbx0 + mbx1) * 0.5
        cy = (mby0 + mby1) * 0.5
        ii4 = ii * 4
        tx0 = plsc.load_gather(tbox_v, [ii4])
        ty0 = plsc.load_gather(tbox_v, [ii4 + 1])
        tx1 = plsc.load_gather(tbox_v, [ii4 + 2])
        ty1 = plsc.load_gather(tbox_v, [ii4 + 3])
        dists = (cx - tx0, cy - ty0, tx1 - cx, ty1 - cy)

        fgl_s = jnp.float32(0.0)
        for side in range(4):
            d = jnp.clip(dists[side] * 32.0, 0.0, 32.0 - 0.01)
            il = d.astype(jnp.int32)
            fl = il.astype(jnp.float32)
            wl = fl + 1.0 - d
            wr = d - fl
            base = _splat_i(side * _REG)

            def max_body(kk, mx):
                v = plsc.load_gather(corner_v, [srcvec, base + kk])
                return jnp.maximum(mx, v)

            mx = lax.fori_loop(0, _REG, max_body,
                               jnp.full((_LANES,), -_BIG, jnp.float32))

            def se_body(kk, sm):
                v = plsc.load_gather(corner_v, [srcvec, base + kk])
                return sm + jnp.exp(v - mx)

            se = lax.fori_loop(0, _REG, se_body, zf)
            lse = mx + _log(se)
            pil = plsc.load_gather(corner_v, [srcvec, base + il])
            pir = plsc.load_gather(corner_v, [srcvec, base + il + 1])
            fgl_s = fgl_s + jnp.sum(lse - wl * pil - wr * pir)

        lane_eq = ii == lyr
        part_v[pl.ds(0, _LANES)] = part_v[pl.ds(0, _LANES)] + jnp.where(lane_eq, vfl_s, 0.0)
        part_v[pl.ds(_LANES, _LANES)] = part_v[pl.ds(_LANES, _LANES)] + jnp.where(lane_eq, fgl_s, 0.0)

    do_task(gwid)

    @pl.when(gwid < _LANES)
    def _second():
        do_task(gwid + 32)

    pltpu.sync_copy(part_v, shared_sp.at[pl.ds(sub * 2 * _LANES, 2 * _LANES)])
    plsc.subcore_barrier()

    @pl.when(sub == 0)
    def _reduce():
        pltpu.sync_copy(shared_sp, red_v)

        def red_body(i, c):
            v, f = c
            return (v + red_v[pl.ds(i * 2 * _LANES, _LANES)],
                    f + red_v[pl.ds(i * 2 * _LANES + _LANES, _LANES)])

        vt, ft = lax.fori_loop(0, _LANES, red_body, (zf, zf))
        vt = vt * (1.0 / 128.0)
        ft = ft * (0.15 / 512.0)
        ii2 = _iota()
        m6 = ii2 < 6
        out_v[...] = zf
        plsc.store_scatter(out_v, [jnp.where(m6, 2 * ii2, 0)], vt, mask=m6)
        plsc.store_scatter(out_v, [jnp.where(m6, 2 * ii2 + 1, 1)], ft, mask=m6)
        pltpu.sync_copy(out_v, out_hbm.at[core])


def kernel(pred_logits, pred_boxes, pred_corners, aux_logits, aux_boxes,
           aux_corners, target_labels, target_boxes):
    plog = pred_logits
    pbox = pred_boxes.reshape(_B, _N * 4)
    pcor = pred_corners
    alog = aux_logits.reshape(5 * _B, _N, _C)
    abox = aux_boxes.reshape(5 * _B, _N * 4)
    acor = aux_corners.reshape(5 * _B, _N, _D)

    mesh = plsc.VectorSubcoreMesh(core_axis_name="c", subcore_axis_name="s",
                                  num_cores=2, num_subcores=16)
    f32 = jnp.float32
    run = pl.kernel(
        _sc_body,
        out_type=jax.ShapeDtypeStruct((2, _LANES), f32),
        mesh=mesh,
        compiler_params=pltpu.CompilerParams(needs_layout_passes=False, use_tc_tiling_on_sc=True),
        scratch_types=[
            pltpu.VMEM((_N, _C), f32),
            pltpu.VMEM((_N * 4,), f32),
            pltpu.VMEM((_N, _D), f32),
            pltpu.VMEM((_M * _NPAD,), f32),
            pltpu.VMEM((_NPAD,), f32),
            pltpu.VMEM((_M,), jnp.int32),
            pltpu.VMEM((_M * 4,), f32),
            pltpu.VMEM((2 * _LANES,), f32),
            pltpu.VMEM((_LANES,), f32),
            pltpu.VMEM_SHARED((_LANES * 2 * _LANES,), f32),
            pltpu.VMEM((_LANES * 2 * _LANES,), f32),
        ],
    )
    out = run(plog, pbox, pcor, alog, abox, acor, target_labels,
              target_boxes.reshape(_B, _M * 4))
    return (out[0] + out[1])[:12]


# speedup vs baseline: 2.4134x; 2.4134x over previous
import jax
import jax.numpy as jnp
from jax import lax
from jax.experimental import pallas as pl
from jax.experimental.pallas import tpu as pltpu
from jax.experimental.pallas import tpu_sc as plsc

_C = 80
_REG = 33
_D = 4 * _REG
_N = 300
_M = 16
_B = 8
_LANES = 16
_CHUNKS = 19
_NPAD = _CHUNKS * _LANES
_BIG = 1e30
_LN2 = 0.6931471805599453


def _iota():
    return lax.iota(jnp.int32, _LANES)


def _splat_i(x):
    return jnp.zeros((_LANES,), jnp.int32) + x


def _sigmoid(x):
    return 1.0 / (1.0 + jnp.exp(-x))


def _log1p_small(z):
    s = z / (2.0 + z)
    s2 = s * s
    return s * (2.0 + s2 * (2.0 / 3.0 + s2 * (2.0 / 5.0 + s2 * (
        2.0 / 7.0 + s2 * (2.0 / 9.0 + s2 * (2.0 / 11.0))))))


def _log(x):
    xi = lax.bitcast_convert_type(x, jnp.int32)
    e = jnp.right_shift(xi, 23) - 127
    m = lax.bitcast_convert_type(
        jnp.bitwise_or(jnp.bitwise_and(xi, 0x7FFFFF), 0x3F800000), jnp.float32)
    big = m > 1.4142135623730951
    m = jnp.where(big, m * 0.5, m)
    ef = (e + big.astype(jnp.int32)).astype(jnp.float32)
    s = (m - 1.0) / (m + 1.0)
    s2 = s * s
    poly = s * (2.0 + s2 * (2.0 / 3.0 + s2 * (2.0 / 5.0 + s2 * (
        2.0 / 7.0 + s2 * (2.0 / 9.0)))))
    return ef * _LN2 + poly


def _sc_body(plog_hbm, pbox_hbm, pcor_hbm, alog_hbm, abox_hbm, acor_hbm,
             labels_hbm, tbox_hbm, out_hbm,
             logit_v, box_v, corner_v, cost_v, pen_v, lab_v, tbox_v,
             part_v, out_v, shared_sp, red_v):
    core = lax.axis_index("c")
    sub = lax.axis_index("s")
    gwid = sub * 2 + core
    ii = _iota()
    zf = jnp.zeros((_LANES,), jnp.float32)

    part_v[pl.ds(0, _LANES)] = zf
    part_v[pl.ds(_LANES, _LANES)] = zf

    def do_task(t):
        lyr = t // _B
        b = t - lyr * _B

        @pl.when(t < _B)
        def _load_final():
            pltpu.sync_copy(plog_hbm.at[b], logit_v)
            pltpu.sync_copy(pbox_hbm.at[b], box_v)
            pltpu.sync_copy(pcor_hbm.at[:, b], corner_v)

        @pl.when(t >= _B)
        def _load_aux():
            la = lyr - 1
            pltpu.sync_copy(alog_hbm.at[la, b], logit_v)
            pltpu.sync_copy(abox_hbm.at[la, b], box_v)
            pltpu.sync_copy(acor_hbm.at[la, :, b], corner_v)

        pltpu.sync_copy(labels_hbm.at[b], lab_v)
        pltpu.sync_copy(tbox_hbm.at[b], tbox_v)

        @pl.loop(0, _CHUNKS)
        def _cost_chunk(c):
            n_idx = c * _LANES + ii
            valid = n_idx < _N
            nc = jnp.minimum(n_idx, _N - 1)
            bx0 = box_v[0, pl.ds(c * _LANES, _LANES)]
            by0 = box_v[1, pl.ds(c * _LANES, _LANES)]
            bx1 = box_v[2, pl.ds(c * _LANES, _LANES)]
            by1 = box_v[3, pl.ds(c * _LANES, _LANES)]

            @pl.loop(0, _M)
            def _cost_m(m):
                ms = _splat_i(m)
                tx0 = plsc.load_gather(tbox_v, [_splat_i(0), ms])
                ty0 = plsc.load_gather(tbox_v, [_splat_i(1), ms])
                tx1 = plsc.load_gather(tbox_v, [_splat_i(2), ms])
                ty1 = plsc.load_gather(tbox_v, [_splat_i(3), ms])
                labm = plsc.load_gather(lab_v, [ms])
                lg = plsc.load_gather(logit_v, [labm, nc])
                cb = (jnp.abs(bx0 - tx0) + jnp.abs(by0 - ty0)
                      + jnp.abs(bx1 - tx1) + jnp.abs(by1 - ty1))
                cv = 5.0 * cb - _sigmoid(lg)
                cv = jnp.where(valid, cv, _BIG)
                cost_v[pl.ds(m * _NPAD + c * _LANES, _LANES)] = cv

        @pl.loop(0, _CHUNKS)
        def _pen_init(c):
            pen_v[pl.ds(c * _LANES, _LANES)] = zf

        def greedy_body(j, srcvec):
            def chunk_body(c, mc):
                mv, mi = mc
                v = (cost_v[pl.ds(j * _NPAD + c * _LANES, _LANES)]
                     + pen_v[pl.ds(c * _LANES, _LANES)])
                idxv = c * _LANES + ii
                take = v < mv
                return (jnp.where(take, v, mv), jnp.where(take, idxv, mi))

            mv, mi = lax.fori_loop(
                0, _CHUNKS, chunk_body,
                (jnp.full((_LANES,), 1e33, jnp.float32),
                 jnp.zeros((_LANES,), jnp.int32)))
            gmin = jnp.min(mv)
            cand = jnp.where(mv == gmin, mi, jnp.int32(2**31 - 1))
            gidx = jnp.min(cand)
            plsc.store_scatter(pen_v, [_splat_i(gidx)],
                               jnp.full((_LANES,), _BIG, jnp.float32),
                               mask=ii == 0)
            return jnp.where(ii == j, gidx, srcvec)

        srcvec = lax.fori_loop(0, _M, greedy_body,
                               jnp.zeros((_LANES,), jnp.int32))

        def vfl_col(cidx, acc):
            l = plsc.load_gather(logit_v, [_splat_i(cidx), srcvec])
            p = _sigmoid(l)
            bce0 = jnp.maximum(l, 0.0) + _log1p_small(jnp.exp(-jnp.abs(l)))
            return acc + (0.75 * p * p) * bce0

        acc0 = lax.fori_loop(0, _C, vfl_col, zf)
        labvec = lab_v[...]
        lt = plsc.load_gather(logit_v, [labvec, srcvec])
        pt = _sigmoid(lt)
        bce0t = jnp.maximum(lt, 0.0) + _log1p_small(jnp.exp(-jnp.abs(lt)))
        vfl_vec = acc0 + (bce0t - lt) - (0.75 * pt * pt) * bce0t
        vfl_s = jnp.sum(vfl_vec)

        mbx0 = plsc.load_gather(box_v, [_splat_i(0), srcvec])
        mby0 = plsc.load_gather(box_v, [_splat_i(1), srcvec])
        mbx1 = plsc.load_gather(box_v, [_splat_i(2), srcvec])
        mby1 = plsc.load_gather(box_v, [_splat_i(3), srcvec])
        cx = (mbx0 + mbx1) * 0.5
        cy = (mby0 + mby1) * 0.5
        tx0 = tbox_v[0, :]
        ty0 = tbox_v[1, :]
        tx1 = tbox_v[2, :]
        ty1 = tbox_v[3, :]
        dists = (cx - tx0, cy - ty0, tx1 - cx, ty1 - cy)

        fgl_s = jnp.float32(0.0)
        for side in range(4):
            d = jnp.clip(dists[side] * 32.0, 0.0, 32.0 - 0.01)
            il = d.astype(jnp.int32)
            fl = il.astype(jnp.float32)
            wl = fl + 1.0 - d
            wr = d - fl
            base = _splat_i(side * _REG)

            def max_body(kk, mx):
                v = plsc.load_gather(corner_v, [base + kk, srcvec])
                return jnp.maximum(mx, v)

            mx = lax.fori_loop(0, _REG, max_body,
                               jnp.full((_LANES,), -_BIG, jnp.float32))

            def se_body(kk, sm):
                v = plsc.load_gather(corner_v, [base + kk, srcvec])
                return sm + jnp.exp(v - mx)

            se = lax.fori_loop(0, _REG, se_body, zf)
            lse = mx + _log(se)
            pil = plsc.load_gather(corner_v, [base + il, srcvec])
            pir = plsc.load_gather(corner_v, [base + il + 1, srcvec])
            fgl_s = fgl_s + jnp.sum(lse - wl * pil - wr * pir)

        lane_eq = ii == lyr
        part_v[pl.ds(0, _LANES)] = (part_v[pl.ds(0, _LANES)]
                                    + jnp.where(lane_eq, vfl_s, 0.0))
        part_v[pl.ds(_LANES, _LANES)] = (part_v[pl.ds(_LANES, _LANES)]
                                         + jnp.where(lane_eq, fgl_s, 0.0))

    do_task(gwid)

    @pl.when(gwid < _LANES)
    def _second():
        do_task(gwid + 32)

    pltpu.sync_copy(part_v, shared_sp.at[pl.ds(sub * 2 * _LANES, 2 * _LANES)])
    plsc.subcore_barrier()

    @pl.when(sub == 0)
    def _reduce():
        pltpu.sync_copy(shared_sp, red_v)

        def red_body(i, c):
            v, f = c
            return (v + red_v[pl.ds(i * 2 * _LANES, _LANES)],
                    f + red_v[pl.ds(i * 2 * _LANES + _LANES, _LANES)])

        vt, ft = lax.fori_loop(0, _LANES, red_body, (zf, zf))
        vt = vt * (1.0 / 128.0)
        ft = ft * (0.15 / 512.0)
        ii2 = _iota()
        m6 = ii2 < 6
        out_v[...] = jnp.zeros((_LANES,), jnp.float32)
        plsc.store_scatter(out_v, [jnp.where(m6, 2 * ii2, 0)], vt, mask=m6)
        plsc.store_scatter(out_v, [jnp.where(m6, 2 * ii2 + 1, 1)], ft, mask=m6)
        pltpu.sync_copy(out_v, out_hbm.at[core])


def kernel(pred_logits, pred_boxes, pred_corners, aux_logits, aux_boxes,
           aux_corners, target_labels, target_boxes):
    plog = jnp.swapaxes(pred_logits, 1, 2)
    pbox = jnp.swapaxes(pred_boxes, 1, 2)
    pcor = jnp.swapaxes(jnp.swapaxes(pred_corners, 1, 2), 0, 1)
    alog = jnp.swapaxes(aux_logits, 2, 3)
    abox = jnp.swapaxes(aux_boxes, 2, 3)
    acor = jnp.swapaxes(jnp.swapaxes(aux_corners, 2, 3), 1, 2)
    tbox = jnp.swapaxes(target_boxes, 1, 2)

    mesh = plsc.VectorSubcoreMesh(core_axis_name="c", subcore_axis_name="s",
                                  num_cores=2, num_subcores=16)
    f32 = jnp.float32
    run = pl.kernel(
        _sc_body,
        out_type=jax.ShapeDtypeStruct((2, _LANES), f32),
        mesh=mesh,
        compiler_params=pltpu.CompilerParams(needs_layout_passes=False,
                                             use_tc_tiling_on_sc=True),
        scratch_types=[
            pltpu.VMEM((_C, _N), f32),
            pltpu.VMEM((4, _N), f32),
            pltpu.VMEM((_D, _N), f32),
            pltpu.VMEM((_M * _NPAD,), f32),
            pltpu.VMEM((_NPAD,), f32),
            pltpu.VMEM((_M,), jnp.int32),
            pltpu.VMEM((4, _M), f32),
            pltpu.VMEM((2 * _LANES,), f32),
            pltpu.VMEM((_LANES,), f32),
            pltpu.VMEM_SHARED((_LANES * 2 * _LANES,), f32),
            pltpu.VMEM((_LANES * 2 * _LANES,), f32),
        ],
    )
    out = run(plog, pbox, pcor, alog, abox, acor, target_labels, tbox)
    return (out[0] + out[1])[:12]
